# Initial kernel scaffold; baseline (speedup 1.0000x reference)
import jax
import jax.numpy as jnp
from jax.experimental import pallas as pl


def kernel(inputs, alpha, delta, root):
    raise NotImplementedError("write your pallas kernel here")


# speedup vs baseline: 18.9927x; 18.9927x over previous
import functools

import jax
import jax.numpy as jnp
import numpy as np
from jax.experimental import pallas as pl
from jax.experimental.pallas import tpu as pltpu

_SMOOTH = 0.025
_FLOOR = 1e-06
_L = 256
_BB = 16


def _pcen_body(x_ref, tri_ref, a_ref, d_ref, r_ref, o_ref, carry_ref):
    t = pl.program_id(1)

    @pl.when(t == 0)
    def _init():
        carry_ref[...] = x_ref[:, 0, :]

    x = x_ref[...]
    sx = _SMOOTH * x
    tri = tri_ref[...]
    dcol = (1.0 - _SMOOTH) * tri[:, 0:1]

    alpha_c = jnp.minimum(a_ref[...], 1.0)
    oor = 1.0 / jnp.maximum(r_ref[...], 1.0)
    delta = d_ref[...]
    t3 = jnp.exp2(oor * jnp.log2(delta))

    for b in range(x.shape[0]):
        m_b = jax.lax.dot(tri, sx[b],
                          precision=jax.lax.Precision.HIGHEST,
                          preferred_element_type=jnp.float32)
        m_b = m_b + dcol * carry_ref[b:b + 1, :]
        carry_ref[b:b + 1, :] = m_b[-1:, :]
        inv_t1 = jnp.exp2((-alpha_c) * jnp.log2(_FLOOR + m_b))
        y = x[b] * inv_t1 + delta
        o_ref[b] = jnp.exp2(oor * jnp.log2(y)) - t3


@functools.partial(jax.jit, static_argnames=())
def kernel(inputs, alpha, delta, root):
    B, T, C = inputs.shape
    nt = T // _L
    nb = B // _BB

    i = np.arange(_L)
    expo = i[:, None] - i[None, :]
    tri = np.where(expo >= 0, (1.0 - _SMOOTH) ** np.maximum(expo, 0), 0.0)
    tri = jnp.asarray(tri, dtype=jnp.float32)

    a2 = alpha.reshape(1, C).astype(jnp.float32)
    d2 = delta.reshape(1, C).astype(jnp.float32)
    r2 = root.reshape(1, C).astype(jnp.float32)

    return pl.pallas_call(
        _pcen_body,
        out_shape=jax.ShapeDtypeStruct((B, T, C), jnp.float32),
        grid=(nb, nt),
        in_specs=[
            pl.BlockSpec((_BB, _L, C), lambda ib, it: (ib, it, 0)),
            pl.BlockSpec((_L, _L), lambda ib, it: (0, 0)),
            pl.BlockSpec((1, C), lambda ib, it: (0, 0)),
            pl.BlockSpec((1, C), lambda ib, it: (0, 0)),
            pl.BlockSpec((1, C), lambda ib, it: (0, 0)),
        ],
        out_specs=pl.BlockSpec((_BB, _L, C), lambda ib, it: (ib, it, 0)),
        scratch_shapes=[pltpu.VMEM((_BB, C), jnp.float32)],
        compiler_params=pltpu.CompilerParams(
            dimension_semantics=("parallel", "arbitrary"),
        ),
        name="pcen_fused",
    )(inputs, tri, a2, d2, r2)
